# baseline (speedup 1.0000x reference)
import functools
import jax
import jax.numpy as jnp
from jax import lax
from jax.experimental import pallas as pl
from jax.experimental.pallas import tpu as pltpu
from jax.experimental.pallas import tpu_sc as plsc

NUM_SEGMENTS = 16
D_MODEL = 4096

_info = plsc.get_sparse_core_info()
_NC, _NS = _info.num_cores, _info.num_subcores
_NW = _NC * _NS

_B = 4 * 8192
_BPW = _B // _NW
_C = 8
_NCHUNK = _BPW // _C
_NPAIR = _NCHUNK // 2


def _body(ids_hbm, table_hbm, out_hbm, idx_v, rows0, rows1, g0, g1, s0, s1):
    wid = lax.axis_index("s") * _NC + lax.axis_index("c")
    base = wid * _BPW
    pltpu.sync_copy(ids_hbm.at[wid], idx_v)

    def gather(j, buf, sem):
        return pltpu.make_async_copy(table_hbm.at[idx_v.at[j]], buf, sem)

    def scatter(j, buf, sem):
        return pltpu.make_async_copy(buf, out_hbm.at[pl.ds(base + j * _C, _C)], sem)

    gather(0, rows0, g0).start()

    def pair(jj, carry):
        j0 = 2 * jj
        j1 = j0 + 1
        @pl.when(jj > 0)
        def _():
            scatter(j0 - 1, rows1, s1).wait()
        gather(j1, rows1, g1).start()
        gather(j0, rows0, g0).wait()
        scatter(j0, rows0, s0).start()
        scatter(j0, rows0, s0).wait()
        @pl.when(jj < _NPAIR - 1)
        def _():
            gather(j1 + 1, rows0, g0).start()
        gather(j1, rows1, g1).wait()
        scatter(j1, rows1, s1).start()
        return carry

    lax.fori_loop(0, _NPAIR, pair, 0)
    scatter(_NCHUNK - 1, rows1, s1).wait()


def kernel(segment_ids, table):
    ids = segment_ids.reshape(_NW, _NCHUNK, _C).astype(jnp.int32)
    run = functools.partial(
        pl.kernel,
        mesh=plsc.VectorSubcoreMesh(core_axis_name="c", subcore_axis_name="s"),
        out_type=jax.ShapeDtypeStruct((_B, D_MODEL), jnp.float32),
        scratch_types=[
            pltpu.VMEM((_NCHUNK, _C), jnp.int32),
            pltpu.VMEM((_C, D_MODEL), jnp.float32),
            pltpu.VMEM((_C, D_MODEL), jnp.float32),
            pltpu.SemaphoreType.DMA,
            pltpu.SemaphoreType.DMA,
            pltpu.SemaphoreType.DMA,
            pltpu.SemaphoreType.DMA,
        ],
    )(_body)
    out = run(ids, table)
    return out.reshape(segment_ids.shape[0], segment_ids.shape[1], D_MODEL)


# speedup vs baseline: 3.8359x; 3.8359x over previous
import functools
import jax
import jax.numpy as jnp
from jax import lax
from jax.experimental import pallas as pl
from jax.experimental.pallas import tpu as pltpu
from jax.experimental.pallas import tpu_sc as plsc

NUM_SEGMENTS = 16
D_MODEL = 4096

_info = plsc.get_sparse_core_info()
_NC, _NS = _info.num_cores, _info.num_subcores
_NW = _NC * _NS
_L = 16

_B = 4 * 8192
_BPW = _B // _NW
_G = _BPW // _L
_GPAIR = _G // 2


def _body(ids_hbm, table_hbm, out_hbm, idx_v, tab_v, sem0, sem1):
    wid = lax.axis_index("s") * _NC + lax.axis_index("c")
    base = wid * _BPW
    pltpu.sync_copy(ids_hbm.at[wid], idx_v)
    pltpu.sync_copy(table_hbm, tab_v)
    lanes = lax.iota(jnp.int32, _L)

    def issue_group(g, sem):
        v = idx_v[pl.ds(g * _L, _L)]
        for l in range(_L):
            s = jnp.sum(jnp.where(lanes == l, v, 0))
            pltpu.make_async_copy(
                tab_v.at[pl.ds(s, 1)],
                out_hbm.at[pl.ds(base + g * _L + l, 1)],
                sem,
            ).start()

    def drain_group(sem):
        d = pltpu.make_async_copy(
            tab_v.at[pl.ds(0, 1)], out_hbm.at[pl.ds(base, 1)], sem
        )
        for _ in range(_L):
            d.wait()

    def pair(gg, carry):
        @pl.when(gg > 0)
        def _():
            drain_group(sem0)
        issue_group(2 * gg, sem0)
        @pl.when(gg > 0)
        def _():
            drain_group(sem1)
        issue_group(2 * gg + 1, sem1)
        return carry

    lax.fori_loop(0, _GPAIR, pair, 0)
    drain_group(sem0)
    drain_group(sem1)


def kernel(segment_ids, table):
    ids = segment_ids.reshape(_NW, _BPW).astype(jnp.int32)
    run = functools.partial(
        pl.kernel,
        mesh=plsc.VectorSubcoreMesh(core_axis_name="c", subcore_axis_name="s"),
        out_type=jax.ShapeDtypeStruct((_B, D_MODEL), jnp.float32),
        compiler_params=pltpu.CompilerParams(needs_layout_passes=False),
        scratch_types=[
            pltpu.VMEM((_BPW,), jnp.int32),
            pltpu.VMEM((NUM_SEGMENTS, D_MODEL), jnp.float32),
            pltpu.SemaphoreType.DMA,
            pltpu.SemaphoreType.DMA,
        ],
    )(_body)
    out = run(ids, table)
    return out.reshape(segment_ids.shape[0], segment_ids.shape[1], D_MODEL)
